# Initial kernel scaffold; baseline (speedup 1.0000x reference)
import jax
import jax.numpy as jnp
from jax.experimental import pallas as pl


def kernel(x, edge_index, edge_attr, batch, params):
    raise NotImplementedError("write your pallas kernel here")


# speedup vs baseline: 2.2374x; 2.2374x over previous
import functools
import math

import jax
import jax.numpy as jnp
from jax import lax
from jax.experimental import pallas as pl
from jax.experimental.pallas import tpu as pltpu
import jax.experimental.pallas.tpu_sc as plsc

H = 128
NUM_LAYERS = 6
HEADS = 4
B = 64
BQ = 256


def _bdot(a, b):
    return lax.dot_general(a.astype(jnp.bfloat16), b.astype(jnp.bfloat16),
                           (((1,), (0,)), ((), ())),
                           preferred_element_type=jnp.float32)


def _fdot(a, b):
    return lax.dot_general(a, b, (((1,), (0,)), ((), ())),
                           preferred_element_type=jnp.float32,
                           precision=jax.lax.Precision.HIGHEST)


def _encode_body(x_ref, wnt_ref, bn_ref, w1st_ref, w1dt_ref, h_ref, hs_ref, hd_ref):
    h = _fdot(x_ref[...], wnt_ref[...])
    h = h + bn_ref[...]
    h_ref[...] = h
    hs_ref[...] = _bdot(h, w1st_ref[...])
    hd_ref[...] = _bdot(h, w1dt_ref[...])


def _encode(x, wnt, bn, w1st0, w1dt0):
    N = x.shape[0]
    R = 1000
    grid = N // R
    full = lambda s: pl.BlockSpec(s, lambda i: (0, 0))
    return pl.pallas_call(
        _encode_body,
        grid=(grid,),
        in_specs=[
            pl.BlockSpec((R, x.shape[1]), lambda i: (i, 0)),
            full(wnt.shape), full(bn.shape), full(w1st0.shape), full(w1dt0.shape),
        ],
        out_specs=[pl.BlockSpec((R, H), lambda i: (i, 0))] * 3,
        out_shape=[jax.ShapeDtypeStruct((N, H), jnp.float32)] * 3,
    )(x, wnt, bn, w1st0, w1dt0)


def _edge_body(ea_ref, wet_ref, be_ref, w1et_ref, b1_ref, ec_ref):
    eh = _fdot(ea_ref[...], wet_ref[...])
    eh = eh + be_ref[...]
    for l in range(NUM_LAYERS):
        ec_ref[l] = _bdot(eh, w1et_ref[l]) + b1_ref[l]


def _edge_terms(ea, wet, be, w1et, b1):
    E = ea.shape[0]
    R = 1000
    grid = E // R
    full = lambda s: pl.BlockSpec(s, lambda *_: (0,) * len(s))
    return pl.pallas_call(
        _edge_body,
        grid=(grid,),
        in_specs=[
            pl.BlockSpec((R, ea.shape[1]), lambda i: (i, 0)),
            full(wet.shape), full(be.shape), full(w1et.shape), full(b1.shape),
        ],
        out_specs=pl.BlockSpec((NUM_LAYERS, R, H), lambda i: (0, i, 0)),
        out_shape=jax.ShapeDtypeStruct((NUM_LAYERS, E, H), jnp.float32),
    )(ea, wet, be, w1et, b1)


def _deg_sc(dst, zeros_h, ones_c, N):
    E = dst.shape[0]
    info = plsc.get_sparse_core_info()
    NC, NS = info.num_cores, info.num_subcores
    NW = NC * NS
    EPW = E // NW
    C = 80
    NCHUNK = EPW // C
    RPT = (N // NS) // 8 * 8
    EXN = N - NS * RPT
    mesh = plsc.VectorSubcoreMesh(core_axis_name="c", subcore_axis_name="s")

    def body(dst_hbm, zc_hbm, on_hbm, cnt_out, dst_v, ones_v, cnt_sh):
        cid = lax.axis_index("c")
        sid = lax.axis_index("s")
        wid = sid * NC + cid
        row0 = sid * RPT
        pltpu.sync_copy(zc_hbm.at[pl.ds(0, RPT)], cnt_sh.at[pl.ds(row0, RPT), :])
        pltpu.sync_copy(on_hbm, ones_v)

        @pl.when(sid == NS - 1)
        def _zero_tail():
            pltpu.sync_copy(zc_hbm.at[pl.ds(0, EXN)],
                            cnt_sh.at[pl.ds(NS * RPT, EXN), :])

        plsc.subcore_barrier()
        ebase = wid * EPW

        def chunk(j, carry):
            off = ebase + j * C
            pltpu.sync_copy(dst_hbm.at[pl.ds(off, C)], dst_v)
            pltpu.sync_copy(ones_v, cnt_sh.at[dst_v], add=True)
            return carry

        lax.fori_loop(0, NCHUNK, chunk, 0, unroll=False)
        plsc.subcore_barrier()
        pltpu.sync_copy(cnt_sh.at[pl.ds(row0, RPT), :],
                        cnt_out.at[cid, pl.ds(row0, RPT), :])

        @pl.when(sid == NS - 1)
        def _write_tail():
            pltpu.sync_copy(cnt_sh.at[pl.ds(NS * RPT, EXN), :],
                            cnt_out.at[cid, pl.ds(NS * RPT, EXN), :])

    f = pl.kernel(
        body,
        out_type=jax.ShapeDtypeStruct((NC, N, H), jnp.float32),
        mesh=mesh,
        scratch_types=[
            pltpu.VMEM((C,), jnp.int32),
            pltpu.VMEM((C, H), jnp.float32),
            pltpu.VMEM_SHARED((N, H), jnp.float32),
        ],
    )
    return f(dst, zeros_h, ones_c)


def _mp_sc(hs, hd, ec, src, dst, zeros_h):
    N = hs.shape[0]
    E = src.shape[0]
    info = plsc.get_sparse_core_info()
    NC, NS = info.num_cores, info.num_subcores
    NW = NC * NS
    EPW = E // NW
    C = 80
    NCHUNK = EPW // C
    RPT = (N // NS) // 8 * 8
    EXN = N - NS * RPT

    out_type = jax.ShapeDtypeStruct((NC, N, H), jnp.float32)
    scratch = [
        pltpu.VMEM((C,), jnp.int32),
        pltpu.VMEM((C,), jnp.int32),
        pltpu.VMEM((C, H), jnp.float32),
        pltpu.VMEM((C, H), jnp.float32),
        pltpu.VMEM((C, H), jnp.float32),
        pltpu.VMEM_SHARED((N, H), jnp.float32),
        pltpu.SemaphoreType.DMA,
        pltpu.SemaphoreType.DMA,
    ]
    mesh = plsc.VectorSubcoreMesh(core_axis_name="c", subcore_axis_name="s")

    def body(hs_hbm, hd_hbm, ec_hbm, src_hbm, dst_hbm, zh_hbm,
             agg_out, src_v, dst_v, hs_v, hd_v, ec_v, agg_sh, sem1, sem2):
        cid = lax.axis_index("c")
        sid = lax.axis_index("s")
        wid = sid * NC + cid
        row0 = sid * RPT
        pltpu.sync_copy(zh_hbm.at[pl.ds(0, RPT)], agg_sh.at[pl.ds(row0, RPT), :])

        @pl.when(sid == NS - 1)
        def _zero_tail():
            pltpu.sync_copy(zh_hbm.at[pl.ds(0, EXN)],
                            agg_sh.at[pl.ds(NS * RPT, EXN), :])

        plsc.subcore_barrier()

        ebase = wid * EPW

        def chunk(j, _):
            off = ebase + j * C
            pltpu.sync_copy(src_hbm.at[pl.ds(off, C)], src_v)
            pltpu.sync_copy(dst_hbm.at[pl.ds(off, C)], dst_v)
            pltpu.async_copy(hs_hbm.at[src_v], hs_v, sem1).wait()
            pltpu.async_copy(hd_hbm.at[dst_v], hd_v, sem2).wait()
            pltpu.sync_copy(ec_hbm.at[pl.ds(off, C), :], ec_v)

            def rbody(r, carry):
                for cc in range(H // 16):
                    sl = pl.ds(cc * 16, 16)
                    v = hs_v[r, sl] + hd_v[r, sl] + ec_v[r, sl]
                    ec_v[r, sl] = jnp.maximum(v, 0.0)
                return carry

            lax.fori_loop(0, C, rbody, 0, unroll=False)
            pltpu.sync_copy(ec_v, agg_sh.at[dst_v], add=True)
            return _

        lax.fori_loop(0, NCHUNK, chunk, 0, unroll=False)
        plsc.subcore_barrier()
        pltpu.sync_copy(agg_sh.at[pl.ds(row0, RPT), :],
                        agg_out.at[cid, pl.ds(row0, RPT), :])

        @pl.when(sid == NS - 1)
        def _write_tail():
            pltpu.sync_copy(agg_sh.at[pl.ds(NS * RPT, EXN), :],
                            agg_out.at[cid, pl.ds(NS * RPT, EXN), :])

    f = pl.kernel(body, out_type=out_type, mesh=mesh, scratch_types=scratch)
    return f(hs, hd, ec, src, dst, zeros_h)


def _gru_body(agg_ref, cnt_ref, h_ref, w2t_ref, b2_ref, wiht_ref, bih_ref,
              whht_ref, bhh_ref, w1st_ref, w1dt_ref, h_out, hs_out, hd_out):
    agg = agg_ref[0] + agg_ref[1]
    deg = cnt_ref[0][:, 0:1] + cnt_ref[1][:, 0:1]
    inp = _fdot(agg, w2t_ref[...])
    inp = inp + deg * b2_ref[...]
    h = h_ref[...]
    gi = _fdot(inp, wiht_ref[...]) + bih_ref[...]
    gh = _fdot(h, whht_ref[...]) + bhh_ref[...]
    r = jax.nn.sigmoid(gi[:, :H] + gh[:, :H])
    z = jax.nn.sigmoid(gi[:, H:2 * H] + gh[:, H:2 * H])
    n = jnp.tanh(gi[:, 2 * H:] + r * gh[:, 2 * H:])
    hn = (1.0 - z) * n + z * h
    h_out[...] = hn
    hs_out[...] = _bdot(hn, w1st_ref[...])
    hd_out[...] = _bdot(hn, w1dt_ref[...])


def _gru(agg2, cnt2, h, w2t, b2, wiht, bih, whht, bhh, w1st, w1dt):
    N = h.shape[0]
    R = 1000
    grid = N // R
    full = lambda s: pl.BlockSpec(s, lambda *_: (0,) * len(s))
    return pl.pallas_call(
        _gru_body,
        grid=(grid,),
        in_specs=[
            pl.BlockSpec((2, R, H), lambda i: (0, i, 0)),
            pl.BlockSpec((2, R, H), lambda i: (0, i, 0)),
            pl.BlockSpec((R, H), lambda i: (i, 0)),
            full(w2t.shape), full(b2.shape), full(wiht.shape), full(bih.shape),
            full(whht.shape), full(bhh.shape), full(w1st.shape), full(w1dt.shape),
        ],
        out_specs=[pl.BlockSpec((R, H), lambda i: (i, 0))] * 3,
        out_shape=[jax.ShapeDtypeStruct((N, H), jnp.float32)] * 3,
    )(agg2, cnt2, h, w2t, b2, wiht, bih, whht, bhh, w1st, w1dt)


def _head_body(h_ref, w1t_ref, b1_ref, w2t_ref, b2_ref, o_ref):
    hid = _fdot(h_ref[...], w1t_ref[...])
    hid = jnp.maximum(hid + b1_ref[...], 0.0)
    o_ref[...] = _fdot(hid, w2t_ref[...]) + b2_ref[...]


def _head(h, w1t, b1, w2t, b2, rows_per_block):
    N = h.shape[0]
    O = w2t.shape[1]
    R = rows_per_block
    grid = N // R
    full = lambda s: pl.BlockSpec(s, lambda *_: (0,) * len(s))
    return pl.pallas_call(
        _head_body,
        grid=(grid,),
        in_specs=[
            pl.BlockSpec((R, h.shape[1]), lambda i: (i, 0)),
            full(w1t.shape), full(b1.shape), full(w2t.shape), full(b2.shape),
        ],
        out_specs=pl.BlockSpec((R, O), lambda i: (i, 0)),
        out_shape=jax.ShapeDtypeStruct((N, O), jnp.float32),
    )(h, w1t, b1, w2t, b2)


def _se_body(b_ref, se_ref):
    b = b_ref[...]
    i = lax.broadcasted_iota(jnp.int32, (b.shape[0], B), 1)
    se_ref[0, :] = jnp.sum((b < i).astype(jnp.int32), axis=0)
    se_ref[1, :] = jnp.sum((b <= i).astype(jnp.int32), axis=0)


def _seg_ranges(batch):
    N = batch.shape[0]
    return pl.pallas_call(
        _se_body,
        in_specs=[pl.BlockSpec((N, 1), lambda: (0, 0))],
        out_specs=pl.BlockSpec((2, B), lambda: (0, 0)),
        out_shape=jax.ShapeDtypeStruct((2, B), jnp.int32),
    )(batch.reshape(N, 1))


def _qkv_body(t_ref, wt_ref, b_ref, q_ref, k_ref, v_ref):
    qkv = _fdot(t_ref[...], wt_ref[...])
    qkv = qkv + b_ref[...]
    q_ref[...] = qkv[:, :H]
    k_ref[...] = qkv[:, H:2 * H]
    v_ref[...] = qkv[:, 2 * H:]


def _qkv(tok, wt, b):
    T = tok.shape[0]
    grid = T // BQ
    full = lambda s: pl.BlockSpec(s, lambda *_: (0, 0))
    return pl.pallas_call(
        _qkv_body,
        grid=(grid,),
        in_specs=[pl.BlockSpec((BQ, H), lambda i: (i, 0)), full(wt.shape), full(b.shape)],
        out_specs=[pl.BlockSpec((BQ, H), lambda i: (i, 0))] * 3,
        out_shape=[jax.ShapeDtypeStruct((T, H), jnp.float32)] * 3,
    )(tok, wt, b)


def _attn_body(q_ref, k_ref, v_ref, tok_ref, segq_ref, segf_ref, se_ref,
               owt_ref, ob_ref, g_ref, be_ref, out_ref):
    NKB = segf_ref.shape[0]
    segq = segq_ref[0, 0]
    qb = q_ref[...]
    starts = se_ref[0]
    ends = se_ref[1]
    iota = lax.iota(jnp.int32, B)
    s_lo = jnp.min(jnp.where(segq < 0, B - 1, segq))
    s_hi = jnp.maximum(jnp.max(segq), 0)
    k_lo = B + jnp.min(jnp.where(iota == s_lo, starts, jnp.int32(1 << 30)))
    k_hi = B + jnp.max(jnp.where(iota == s_hi, ends, jnp.int32(0)))
    kb_lo = jnp.maximum(k_lo // BQ, 1)
    kb_hi = jnp.minimum((k_hi - 1) // BQ, NKB - 1)
    scale = 1.0 / math.sqrt(H // HEADS)
    D = H // HEADS

    def process(j, carry):
        M, L, ACC = carry
        kb = k_ref[pl.ds(j * BQ, BQ), :]
        vb = v_ref[pl.ds(j * BQ, BQ), :]
        segk = segf_ref[j, 0]
        mask = segq[:, None] == segk[None, :]
        Ms, Ls, As = [], [], []
        for hh in range(HEADS):
            qh = qb[:, hh * D:(hh + 1) * D]
            kh = kb[:, hh * D:(hh + 1) * D]
            vh = vb[:, hh * D:(hh + 1) * D]
            lg = lax.dot_general(qh, kh, (((1,), (1,)), ((), ())),
                                 preferred_element_type=jnp.float32,
                                 precision=jax.lax.Precision.HIGHEST) * scale
            m_h = M[:, hh]
            rowmax = jnp.max(jnp.where(mask, lg, -1e30), axis=1)
            m_new = jnp.maximum(m_h, rowmax)
            p = jnp.where(mask, jnp.exp(lg - m_new[:, None]), 0.0)
            alpha = jnp.exp(m_h - m_new)
            l_new = alpha * L[:, hh] + jnp.sum(p, axis=1)
            acc_h = ACC[:, hh * D:(hh + 1) * D]
            acc_new = alpha[:, None] * acc_h + _fdot(p, vh)
            Ms.append(m_new)
            Ls.append(l_new)
            As.append(acc_new)
        return (jnp.stack(Ms, 1), jnp.stack(Ls, 1), jnp.concatenate(As, 1))

    init = (jnp.full((BQ, HEADS), -1e30, jnp.float32),
            jnp.zeros((BQ, HEADS), jnp.float32),
            jnp.zeros((BQ, H), jnp.float32))
    carry = process(0, init)
    M, L, ACC = lax.fori_loop(kb_lo, kb_hi + 1, process, carry)
    L = jnp.where(L > 0.0, L, 1.0)
    o = jnp.concatenate(
        [ACC[:, hh * D:(hh + 1) * D] / L[:, hh][:, None] for hh in range(HEADS)], 1)
    att = _fdot(o, owt_ref[...]) + ob_ref[...]
    res = tok_ref[...] + att
    mu = jnp.mean(res, axis=1, keepdims=True)
    var = jnp.mean((res - mu) ** 2, axis=1, keepdims=True)
    out_ref[...] = (res - mu) / jnp.sqrt(var + 1e-5) * g_ref[...] + be_ref[...]


def _attn(q, k, v, tok, seg2d, se, owt, ob, g, be):
    T = tok.shape[0]
    grid = T // BQ
    full = lambda s: pl.BlockSpec(s, lambda *_: (0,) * len(s))
    return pl.pallas_call(
        _attn_body,
        grid=(grid,),
        in_specs=[
            pl.BlockSpec((BQ, H), lambda i: (i, 0)),
            full(k.shape), full(v.shape),
            pl.BlockSpec((BQ, H), lambda i: (i, 0)),
            pl.BlockSpec((1, 1, BQ), lambda i: (i, 0, 0)),
            full(seg2d.shape), full(se.shape),
            full(owt.shape), full(ob.shape), full(g.shape), full(be.shape),
        ],
        out_specs=pl.BlockSpec((BQ, H), lambda i: (i, 0)),
        out_shape=jax.ShapeDtypeStruct((T, H), jnp.float32),
    )(q, k, v, tok, seg2d, seg2d, se, owt, ob, g, be)


def kernel(x, edge_index, edge_attr, batch, params):
    N = x.shape[0]
    E = edge_index.shape[1]
    src = edge_index[0]
    dst = edge_index[1]

    wn, bn = params['node_enc']
    we, be = params['edge_enc']
    layers = params['layers']

    w1st = jnp.stack([lp['w1'][:, :H].T for lp in layers])
    w1dt = jnp.stack([lp['w1'][:, H:2 * H].T for lp in layers])
    w1et = jnp.stack([lp['w1'][:, 2 * H:].T for lp in layers])
    b1 = jnp.stack([lp['b1'].reshape(1, H) for lp in layers])

    h, hs, hd = _encode(x, wn.T, bn.reshape(1, H), w1st[0], w1dt[0])
    ec = _edge_terms(edge_attr, we.T, be.reshape(1, H), w1et, b1)

    zeros_h = jnp.zeros((N // 16, H), jnp.float32)
    ones_c = jnp.ones((80, H), jnp.float32)

    cnt2 = _deg_sc(dst, zeros_h, ones_c, N)
    for l in range(NUM_LAYERS):
        lp = layers[l]
        agg2 = _mp_sc(hs, hd, ec[l], src, dst, zeros_h)
        nxt = min(l + 1, NUM_LAYERS - 1)
        h, hs, hd = _gru(agg2, cnt2, h,
                         lp['w2'].T, lp['b2'].reshape(1, H),
                         lp['w_ih'].T, lp['b_ih'].reshape(1, 3 * H),
                         lp['w_hh'].T, lp['b_hh'].reshape(1, 3 * H),
                         w1st[nxt], w1dt[nxt])

    nh = params['node_head']
    node_preds = _head(h, nh['w1'].T, nh['b1'].reshape(1, -1),
                       nh['w2'].T, nh['b2'].reshape(1, -1), 1000)

    g = params['glob']
    T = B + N
    Tp = ((T + BQ - 1) // BQ) * BQ
    pad = Tp - T
    pool = jnp.tile(g['pool'][0], (B, 1))
    tok = jnp.concatenate([pool, h, jnp.zeros((pad, H), jnp.float32)], axis=0)
    seg = jnp.concatenate([jnp.arange(B, dtype=jnp.int32),
                           batch.astype(jnp.int32),
                           jnp.full((pad,), -1, jnp.int32)])
    seg2d = seg.reshape(Tp // BQ, 1, BQ)
    se = _seg_ranges(batch.astype(jnp.int32))

    for key_attn, key_ln in (('attn1', 'ln1'), ('attn2', 'ln2')):
        in_w, in_b, out_w, out_b = g[key_attn]
        ln_g, ln_b = g[key_ln]
        q, k, v = _qkv(tok, in_w.T, in_b.reshape(1, 3 * H))
        tok = _attn(q, k, v, tok, seg2d, se, out_w.T, out_b.reshape(1, H),
                    ln_g.reshape(1, H), ln_b.reshape(1, H))

    gh = g['head']
    glob = _head(tok[:B], gh['w1'].T, gh['b1'].reshape(1, -1),
                 gh['w2'].T, gh['b2'].reshape(1, -1), B)
    return (node_preds, glob)
